# baseline (speedup 1.0000x reference)
import functools

import jax
import jax.numpy as jnp
from jax.experimental import pallas as pl
from jax.experimental.pallas import tpu as pltpu
from jax.experimental.pallas import tpu_sc as plsc

_N = 4096
_ROWS_PER_WORD = 32
_NUM_WORD_ROWS = _N // _ROWS_PER_WORD
_TC_BLOCK = 256
_TC_GRID = _N // _TC_BLOCK

_NUM_WORKERS = 32
_HALF_WORDS = _NUM_WORD_ROWS * _N // 2
_WORDS_PER_WORKER = _HALF_WORDS // _NUM_WORKERS
_MRG = 4
_NCH = 4

_KS = (0, 42, 0x1BD11BDA ^ 0 ^ 42)
_ROTS = ((13, 15, 26, 6), (17, 29, 16, 24))
_THRESH = 5872026


def _rotl(x, r):
    return (x << jnp.uint32(r)) | (x >> jnp.uint32(32 - r))


def _threefry_keep(flat_u32):
    x0 = jnp.full_like(flat_u32, jnp.uint32(_KS[0]))
    x1 = flat_u32 + jnp.uint32(_KS[1])
    for i in range(5):
        for r in _ROTS[i % 2]:
            x0 = x0 + x1
            x1 = _rotl(x1, r) ^ x0
        x0 = x0 + jnp.uint32(_KS[(i + 1) % 3])
        x1 = x1 + jnp.uint32(_KS[(i + 2) % 3] + i + 1)
    bits = x0 ^ x1
    return (bits >> jnp.uint32(9)) < jnp.uint32(_THRESH)


def _bitmask_kernel(adj_ref, mask_ref):
    riota = jax.lax.broadcasted_iota(jnp.int32, (_ROWS_PER_WORD, _N), 0)
    bitval = jnp.int32(1) << riota
    for w in range(_TC_BLOCK // _ROWS_PER_WORD):
        rows = adj_ref[pl.ds(_ROWS_PER_WORD * w, _ROWS_PER_WORD), :]
        bits = jnp.where(rows != 0.0, bitval, jnp.int32(0))
        mask_ref[w, :] = jnp.sum(bits, axis=0)


def _popcount16(v):
    c55 = jnp.full_like(v, 0x55555555)
    c33 = jnp.full_like(v, 0x33333333)
    c0f = jnp.full_like(v, 0x0F0F0F0F)
    v = v - (jax.lax.shift_right_logical(v, 1) & c55)
    v = (v & c33) + (jax.lax.shift_right_logical(v, 2) & c33)
    v = (v + jax.lax.shift_right_logical(v, 4)) & c0f
    return jax.lax.shift_right_logical(v * 0x01010101, 24)


def _lane_max(v, lane):
    for s in (8, 4, 2, 1):
        v = jnp.maximum(v, v.at[lane ^ s].get(mode="promise_in_bounds"))
    return v[0]


def _sc_drop_body(half, mask_hbm, kept_hbm, maskbuf, wordbuf):
    cid = jax.lax.axis_index("c")
    sid = jax.lax.axis_index("s")
    wid = sid * 2 + cid
    slab_word = wid * _WORDS_PER_WORKER
    base_word = half * _HALF_WORDS + slab_word

    pltpu.sync_copy(mask_hbm.at[pl.ds(base_word, _WORDS_PER_WORKER)], maskbuf)

    lane = jax.lax.iota(jnp.int32, 16)
    w0 = base_word // _N

    grp = _MRG * _NCH

    @pl.loop(0, _WORDS_PER_WORKER // (16 * grp))
    def _vec(i):
        ws = [[] for _ in range(_NCH)]
        fbs = [[] for _ in range(_NCH)]
        bounds = []
        for c in range(_NCH):
            psum = None
            for m in range(_MRG):
                v_idx = i * grp + c * _MRG + m
                w = maskbuf[pl.ds(v_idx * 16, 16)]
                lw = v_idx * 16 + lane
                fbs[c].append((w0 + jax.lax.shift_right_logical(lw, 12))
                              * (32 * _N) + (lw & (_N - 1)))
                ws[c].append(w)
                p = _popcount16(w)
                psum = p if psum is None else psum + p
            bounds.append(_lane_max(psum, lane))
        rounds = bounds[0]
        for b in bounds[1:]:
            rounds = jnp.maximum(rounds, b)

        zero = jnp.zeros((16,), jnp.int32)
        init = sum((tuple(ws[c]) for c in range(_NCH)), ()) + (zero,) * grp

        @pl.loop(0, rounds, init_carry=init)
        def _round(r, carry):
            wr = [list(carry[c * _MRG:(c + 1) * _MRG]) for c in range(_NCH)]
            kept = [list(carry[grp + c * _MRG:grp + (c + 1) * _MRG])
                    for c in range(_NCH)]
            for c in range(_NCH):
                w_ = wr[c]
                fb = fbs[c]
                live = [w_[m] != 0 for m in range(_MRG)]
                w_sel = jnp.where(live[0], w_[0], jnp.where(
                    live[1], w_[1], jnp.where(live[2], w_[2], w_[3])))
                f_sel = jnp.where(live[0], fb[0], jnp.where(
                    live[1], fb[1], jnp.where(live[2], fb[2], fb[3])))
                lsb = w_sel & (0 - w_sel)
                bit = _popcount16(lsb - 1)
                flat = f_sel + (bit << 12)
                keep = _threefry_keep(flat.astype(jnp.uint32))
                upds = [jnp.where(live[0], lsb, 0)]
                upds.append(jnp.where(live[0], 0,
                                      jnp.where(live[1], lsb, 0)))
                upds.append(jnp.where(live[0], 0, jnp.where(
                    live[1], 0, jnp.where(live[2], lsb, 0))))
                upds.append(jnp.where(live[0], 0, jnp.where(
                    live[1], 0, jnp.where(live[2], 0,
                                          jnp.where(live[3], lsb, 0)))))
                for m in range(_MRG):
                    kept[c][m] = kept[c][m] | jnp.where(keep, upds[m], 0)
                    wr[c][m] = wr[c][m] ^ upds[m]
            return (sum((tuple(wr[c]) for c in range(_NCH)), ())
                    + sum((tuple(kept[c]) for c in range(_NCH)), ()))

        carry = _round
        for j in range(grp):
            wordbuf[pl.ds((i * grp + j) * 16, 16)] = carry[grp + j]

    pltpu.sync_copy(wordbuf, kept_hbm.at[pl.ds(slab_word, _WORDS_PER_WORKER)])


def _sc_half(mask_flat, half):
    return pl.kernel(
        functools.partial(_sc_drop_body, half),
        out_type=jax.ShapeDtypeStruct((_HALF_WORDS,), jnp.int32),
        mesh=plsc.VectorSubcoreMesh(core_axis_name="c", subcore_axis_name="s",
                                    num_cores=2, num_subcores=16),
        scratch_types=[
            pltpu.VMEM((_WORDS_PER_WORKER,), jnp.int32),
            pltpu.VMEM((_WORDS_PER_WORKER,), jnp.int32),
        ],
    )(mask_flat)


def _apply_kernel(adj_ref, kept_ref, out_ref):
    riota = jax.lax.broadcasted_iota(jnp.int32, (_ROWS_PER_WORD, _N), 0)
    one = jnp.int32(1)
    for w in range(_TC_BLOCK // _ROWS_PER_WORD):
        rows = adj_ref[pl.ds(_ROWS_PER_WORD * w, _ROWS_PER_WORD), :]
        word = kept_ref[w, :]
        bits = jax.lax.shift_right_logical(
            jnp.broadcast_to(word[None, :], (_ROWS_PER_WORD, _N)), riota) & one
        out_ref[pl.ds(_ROWS_PER_WORD * w, _ROWS_PER_WORD), :] = jnp.where(
            bits != 0, rows, 0.0)


def _apply_kernel_aliased(adj_ref, kept_ref, prev_ref, out_ref):
    del prev_ref
    _apply_kernel(adj_ref, kept_ref, out_ref)


def kernel(x, adj):
    mask = pl.pallas_call(
        _bitmask_kernel,
        grid=(_TC_GRID,),
        in_specs=[pl.BlockSpec((_TC_BLOCK, _N), lambda g: (g, 0))],
        out_specs=pl.BlockSpec((_TC_BLOCK // _ROWS_PER_WORD, _N),
                               lambda g: (g, 0)),
        out_shape=jax.ShapeDtypeStruct((_NUM_WORD_ROWS, _N), jnp.int32),
        compiler_params=pltpu.CompilerParams(
            dimension_semantics=("arbitrary",)),
    )(adj)

    mask_flat = jnp.reshape(mask, (_NUM_WORD_ROWS * _N,))
    kept0 = jnp.reshape(_sc_half(mask_flat, 0), (_NUM_WORD_ROWS // 2, _N))
    kept1 = jnp.reshape(_sc_half(mask_flat, 1), (_NUM_WORD_ROWS // 2, _N))

    half_grid = _TC_GRID // 2
    kept_blk = _TC_BLOCK // _ROWS_PER_WORD

    partial = pl.pallas_call(
        _apply_kernel,
        grid=(half_grid,),
        in_specs=[
            pl.BlockSpec((_TC_BLOCK, _N), lambda g: (g, 0)),
            pl.BlockSpec((kept_blk, _N), lambda g: (g, 0)),
        ],
        out_specs=pl.BlockSpec((_TC_BLOCK, _N), lambda g: (g, 0)),
        out_shape=jax.ShapeDtypeStruct((_N, _N), jnp.float32),
        compiler_params=pltpu.CompilerParams(
            dimension_semantics=("arbitrary",)),
    )(adj, kept0)

    t = pl.pallas_call(
        _apply_kernel_aliased,
        grid=(half_grid,),
        in_specs=[
            pl.BlockSpec((_TC_BLOCK, _N), lambda g: (g + half_grid, 0)),
            pl.BlockSpec((kept_blk, _N), lambda g: (g, 0)),
            pl.BlockSpec(memory_space=pl.ANY),
        ],
        out_specs=pl.BlockSpec((_TC_BLOCK, _N), lambda g: (g + half_grid, 0)),
        out_shape=jax.ShapeDtypeStruct((_N, _N), jnp.float32),
        input_output_aliases={2: 0},
        compiler_params=pltpu.CompilerParams(
            dimension_semantics=("arbitrary",)),
    )(adj, kept1, partial)

    return (x, t)


# speedup vs baseline: 1.0882x; 1.0882x over previous
import functools

import jax
import jax.numpy as jnp
from jax.experimental import pallas as pl
from jax.experimental.pallas import tpu as pltpu
from jax.experimental.pallas import tpu_sc as plsc

_N = 4096
_ROWS_PER_WORD = 32
_NUM_WORD_ROWS = _N // _ROWS_PER_WORD
_TC_BLOCK = 256
_TC_GRID = _N // _TC_BLOCK

_NUM_WORKERS = 32
_HALF_WORDS = _NUM_WORD_ROWS * _N // 2
_WORDS_PER_WORKER = _HALF_WORDS // _NUM_WORKERS
_MRG = 4
_NCH = 2

_KS = (0, 42, 0x1BD11BDA ^ 0 ^ 42)
_ROTS = ((13, 15, 26, 6), (17, 29, 16, 24))
_THRESH = 5872026


def _rotl(x, r):
    return (x << jnp.uint32(r)) | (x >> jnp.uint32(32 - r))


def _threefry_keep(flat_u32):
    x0 = jnp.full_like(flat_u32, jnp.uint32(_KS[0]))
    x1 = flat_u32 + jnp.uint32(_KS[1])
    for i in range(5):
        for r in _ROTS[i % 2]:
            x0 = x0 + x1
            x1 = _rotl(x1, r) ^ x0
        x0 = x0 + jnp.uint32(_KS[(i + 1) % 3])
        x1 = x1 + jnp.uint32(_KS[(i + 2) % 3] + i + 1)
    bits = x0 ^ x1
    return (bits >> jnp.uint32(9)) < jnp.uint32(_THRESH)


def _bitmask_kernel(adj_ref, mask_ref):
    riota = jax.lax.broadcasted_iota(jnp.int32, (_ROWS_PER_WORD, _N), 0)
    bitval = jnp.int32(1) << riota
    for w in range(_TC_BLOCK // _ROWS_PER_WORD):
        rows = adj_ref[pl.ds(_ROWS_PER_WORD * w, _ROWS_PER_WORD), :]
        bits = jnp.where(rows != 0.0, bitval, jnp.int32(0))
        mask_ref[w, :] = jnp.sum(bits, axis=0)


def _popcount16(v):
    c55 = jnp.full_like(v, 0x55555555)
    c33 = jnp.full_like(v, 0x33333333)
    c0f = jnp.full_like(v, 0x0F0F0F0F)
    v = v - (jax.lax.shift_right_logical(v, 1) & c55)
    v = (v & c33) + (jax.lax.shift_right_logical(v, 2) & c33)
    v = (v + jax.lax.shift_right_logical(v, 4)) & c0f
    return jax.lax.shift_right_logical(v * 0x01010101, 24)


def _lane_max(v, lane):
    for s in (8, 4, 2, 1):
        v = jnp.maximum(v, v.at[lane ^ s].get(mode="promise_in_bounds"))
    return v[0]


def _sc_drop_body(half, mask_hbm, kept_hbm, maskbuf, wordbuf):
    cid = jax.lax.axis_index("c")
    sid = jax.lax.axis_index("s")
    wid = sid * 2 + cid
    slab_word = wid * _WORDS_PER_WORKER
    base_word = half * _HALF_WORDS + slab_word

    pltpu.sync_copy(mask_hbm.at[pl.ds(base_word, _WORDS_PER_WORKER)], maskbuf)

    lane = jax.lax.iota(jnp.int32, 16)
    w0 = base_word // _N

    grp = _MRG * _NCH

    @pl.loop(0, _WORDS_PER_WORKER // (16 * grp))
    def _vec(i):
        ws = [[] for _ in range(_NCH)]
        fbs = [[] for _ in range(_NCH)]
        bounds = []
        for c in range(_NCH):
            psum = None
            for m in range(_MRG):
                v_idx = i * grp + c * _MRG + m
                w = maskbuf[pl.ds(v_idx * 16, 16)]
                lw = v_idx * 16 + lane
                fbs[c].append((w0 + jax.lax.shift_right_logical(lw, 12))
                              * (32 * _N) + (lw & (_N - 1)))
                ws[c].append(w)
                p = _popcount16(w)
                psum = p if psum is None else psum + p
            bounds.append(_lane_max(psum, lane))
        rounds = bounds[0]
        for b in bounds[1:]:
            rounds = jnp.maximum(rounds, b)

        zero = jnp.zeros((16,), jnp.int32)
        init = sum((tuple(ws[c]) for c in range(_NCH)), ()) + (zero,) * grp

        @pl.loop(0, rounds, init_carry=init)
        def _round(r, carry):
            wr = [list(carry[c * _MRG:(c + 1) * _MRG]) for c in range(_NCH)]
            kept = [list(carry[grp + c * _MRG:grp + (c + 1) * _MRG])
                    for c in range(_NCH)]
            for c in range(_NCH):
                w_ = wr[c]
                fb = fbs[c]
                live = [w_[m] != 0 for m in range(_MRG)]
                w_sel = jnp.where(live[0], w_[0], jnp.where(
                    live[1], w_[1], jnp.where(live[2], w_[2], w_[3])))
                f_sel = jnp.where(live[0], fb[0], jnp.where(
                    live[1], fb[1], jnp.where(live[2], fb[2], fb[3])))
                lsb = w_sel & (0 - w_sel)
                bit = _popcount16(lsb - 1)
                flat = f_sel + (bit << 12)
                keep = _threefry_keep(flat.astype(jnp.uint32))
                upds = [jnp.where(live[0], lsb, 0)]
                upds.append(jnp.where(live[0], 0,
                                      jnp.where(live[1], lsb, 0)))
                upds.append(jnp.where(live[0], 0, jnp.where(
                    live[1], 0, jnp.where(live[2], lsb, 0))))
                upds.append(jnp.where(live[0], 0, jnp.where(
                    live[1], 0, jnp.where(live[2], 0,
                                          jnp.where(live[3], lsb, 0)))))
                for m in range(_MRG):
                    kept[c][m] = kept[c][m] | jnp.where(keep, upds[m], 0)
                    wr[c][m] = wr[c][m] ^ upds[m]
            return (sum((tuple(wr[c]) for c in range(_NCH)), ())
                    + sum((tuple(kept[c]) for c in range(_NCH)), ()))

        carry = _round
        for j in range(grp):
            wordbuf[pl.ds((i * grp + j) * 16, 16)] = carry[grp + j]

    pltpu.sync_copy(wordbuf, kept_hbm.at[pl.ds(slab_word, _WORDS_PER_WORKER)])


def _sc_half(mask_flat, half):
    return pl.kernel(
        functools.partial(_sc_drop_body, half),
        out_type=jax.ShapeDtypeStruct((_HALF_WORDS,), jnp.int32),
        mesh=plsc.VectorSubcoreMesh(core_axis_name="c", subcore_axis_name="s",
                                    num_cores=2, num_subcores=16),
        scratch_types=[
            pltpu.VMEM((_WORDS_PER_WORKER,), jnp.int32),
            pltpu.VMEM((_WORDS_PER_WORKER,), jnp.int32),
        ],
    )(mask_flat)


def _apply_kernel(adj_ref, kept_ref, out_ref):
    riota = jax.lax.broadcasted_iota(jnp.int32, (_ROWS_PER_WORD, _N), 0)
    one = jnp.int32(1)
    for w in range(_TC_BLOCK // _ROWS_PER_WORD):
        rows = adj_ref[pl.ds(_ROWS_PER_WORD * w, _ROWS_PER_WORD), :]
        word = kept_ref[w, :]
        bits = jax.lax.shift_right_logical(
            jnp.broadcast_to(word[None, :], (_ROWS_PER_WORD, _N)), riota) & one
        out_ref[pl.ds(_ROWS_PER_WORD * w, _ROWS_PER_WORD), :] = jnp.where(
            bits != 0, rows, 0.0)


def _apply_kernel_aliased(adj_ref, kept_ref, prev_ref, out_ref):
    del prev_ref
    _apply_kernel(adj_ref, kept_ref, out_ref)


def kernel(x, adj):
    mask = pl.pallas_call(
        _bitmask_kernel,
        grid=(_TC_GRID,),
        in_specs=[pl.BlockSpec((_TC_BLOCK, _N), lambda g: (g, 0))],
        out_specs=pl.BlockSpec((_TC_BLOCK // _ROWS_PER_WORD, _N),
                               lambda g: (g, 0)),
        out_shape=jax.ShapeDtypeStruct((_NUM_WORD_ROWS, _N), jnp.int32),
        compiler_params=pltpu.CompilerParams(
            dimension_semantics=("arbitrary",)),
    )(adj)

    mask_flat = jnp.reshape(mask, (_NUM_WORD_ROWS * _N,))
    kept0 = jnp.reshape(_sc_half(mask_flat, 0), (_NUM_WORD_ROWS // 2, _N))
    kept1 = jnp.reshape(_sc_half(mask_flat, 1), (_NUM_WORD_ROWS // 2, _N))

    half_grid = _TC_GRID // 2
    kept_blk = _TC_BLOCK // _ROWS_PER_WORD

    partial = pl.pallas_call(
        _apply_kernel,
        grid=(half_grid,),
        in_specs=[
            pl.BlockSpec((_TC_BLOCK, _N), lambda g: (g, 0)),
            pl.BlockSpec((kept_blk, _N), lambda g: (g, 0)),
        ],
        out_specs=pl.BlockSpec((_TC_BLOCK, _N), lambda g: (g, 0)),
        out_shape=jax.ShapeDtypeStruct((_N, _N), jnp.float32),
        compiler_params=pltpu.CompilerParams(
            dimension_semantics=("arbitrary",)),
    )(adj, kept0)

    t = pl.pallas_call(
        _apply_kernel_aliased,
        grid=(half_grid,),
        in_specs=[
            pl.BlockSpec((_TC_BLOCK, _N), lambda g: (g + half_grid, 0)),
            pl.BlockSpec((kept_blk, _N), lambda g: (g, 0)),
            pl.BlockSpec(memory_space=pl.ANY),
        ],
        out_specs=pl.BlockSpec((_TC_BLOCK, _N), lambda g: (g + half_grid, 0)),
        out_shape=jax.ShapeDtypeStruct((_N, _N), jnp.float32),
        input_output_aliases={2: 0},
        compiler_params=pltpu.CompilerParams(
            dimension_semantics=("arbitrary",)),
    )(adj, kept1, partial)

    return (x, t)
